# baseline (speedup 1.0000x reference)
import functools

import jax
import jax.numpy as jnp
from jax import lax
from jax.experimental import pallas as pl
from jax.experimental.pallas import tpu as pltpu
from jax.experimental.pallas import tpu_sc as plsc

T = 2048
D = 2048
E = 8
K = 2
DFF = 1024
DSH = 1024
CAP = 640
S = T * K
EC = E * CAP

NC = 2
NS = 16
NW = NC * NS
LANES = 16

BT = 256
ROWS_PER_W = EC // NW
TOK_PER_W = T // NW
RC = 32
XC = D // 2
NBUF = 3


def _pipelined_gather(src_hbm, idx_v, dst_hbm, dst_base, nrows,
                      bufs, gsems, wsems):
    nb = len(bufs)
    nch = nrows // RC

    def g(cc):
        return pltpu.async_copy(
            src_hbm.at[idx_v.at[pl.ds(cc * RC, RC)]], bufs[cc % nb],
            gsems[cc % nb])

    gd = {}
    wd = {}
    for b in range(min(nb, nch)):
        gd[b] = g(b)
    for cc in range(nch):
        gd[cc].wait()
        wd[cc] = pltpu.async_copy(
            bufs[cc % nb], dst_hbm.at[pl.ds(dst_base + cc * RC, RC)],
            wsems[cc % nb])
        if cc + nb < nch:
            wd[cc].wait()
            gd[cc + nb] = g(cc + nb)
    for cc in range(max(0, nch - nb), nch):
        wd[cc].wait()


def _router_body(x_ref, gwt_ref, slots_ref, ws_ref, xbf_ref, carry_ref):
    i = pl.program_id(0)

    @pl.when(i == 0)
    def _():
        carry_ref[...] = jnp.zeros_like(carry_ref)

    xb = x_ref[...]
    logits = jnp.dot(xb, gwt_ref[...], preferred_element_type=jnp.float32)
    scores = jax.nn.sigmoid(logits)
    lane = lax.broadcasted_iota(jnp.int32, (BT, E), 1)

    m1 = jnp.max(scores, axis=1, keepdims=True)
    e1 = jnp.min(jnp.where(scores == m1, lane, E), axis=1, keepdims=True)
    masked = jnp.where(lane == e1, -1.0, scores)
    m2 = jnp.max(masked, axis=1, keepdims=True)
    e2 = jnp.min(jnp.where(masked == m2, lane, E), axis=1, keepdims=True)

    wsum = m1 + m2 + 1e-20
    w1 = m1 / wsum
    w2 = m2 / wsum

    oh1 = (lane == e1).astype(jnp.float32)
    oh2 = (lane == e2).astype(jnp.float32)
    ohsum = oh1 + oh2

    r = lax.broadcasted_iota(jnp.int32, (BT, BT), 0)
    c = lax.broadcasted_iota(jnp.int32, (BT, BT), 1)
    tril = (c < r).astype(jnp.float32)
    cnt = jnp.dot(tril, ohsum, preferred_element_type=jnp.float32)
    cnt = cnt + carry_ref[...]
    carry_ref[...] = carry_ref[...] + jnp.sum(ohsum, axis=0, keepdims=True)

    pos1 = jnp.sum(jnp.where(oh1 > 0, cnt, 0.0), axis=1, keepdims=True)
    pos2 = jnp.sum(jnp.where(oh2 > 0, cnt, 0.0), axis=1, keepdims=True)
    p1 = pos1.astype(jnp.int32)
    p2 = pos2.astype(jnp.int32)
    v1 = p1 < CAP
    v2 = p2 < CAP
    slot1 = jnp.where(v1, e1 * CAP + p1, EC)
    slot2 = jnp.where(v2, e2 * CAP + p2, EC)
    slots_ref[...] = jnp.concatenate([slot1, slot2], axis=1)
    ws_ref[...] = jnp.concatenate(
        [jnp.where(v1, w1, 0.0), jnp.where(v2, w2, 0.0)], axis=1)
    xbf_ref[...] = xb.astype(jnp.bfloat16)


def _router(x, gwt):
    return pl.pallas_call(
        _router_body,
        grid=(T // BT,),
        in_specs=[
            pl.BlockSpec((BT, D), lambda i: (i, 0)),
            pl.BlockSpec((D, E), lambda i: (0, 0)),
        ],
        out_specs=[
            pl.BlockSpec((BT, K), lambda i: (i, 0)),
            pl.BlockSpec((BT, K), lambda i: (i, 0)),
            pl.BlockSpec((BT, D), lambda i: (i, 0)),
        ],
        out_shape=[
            jax.ShapeDtypeStruct((T, K), jnp.int32),
            jax.ShapeDtypeStruct((T, K), jnp.float32),
            jax.ShapeDtypeStruct((T, D), jnp.bfloat16),
        ],
        scratch_shapes=[pltpu.VMEM((1, E), jnp.float32)],
    )(x, gwt)


_SCAN_UNROLL = 4


def _dispatch_body(slots_hbm, x_hbm, xe_hbm, slots_v, tok_v,
                   b0, b1, b2, g0, g1, g2, w0, w1, w2):
    wid = lax.axis_index("s") * NC + lax.axis_index("c")
    base = wid * ROWS_PER_W

    pltpu.sync_copy(slots_hbm, slots_v)
    for j in range(ROWS_PER_W // LANES):
        tok_v[pl.ds(j * LANES, LANES)] = jnp.zeros((LANES,), jnp.int32)

    iot = lax.iota(jnp.int32, LANES)

    def scan(j, carry):
        for u in range(_SCAN_UNROLL):
            off = j * (LANES * _SCAN_UNROLL) + u * LANES
            idx = slots_v[pl.ds(off, LANES)]
            tok = jnp.right_shift(off + iot, 1)
            loc = idx - base
            mask = (loc >= 0) & (loc < ROWS_PER_W)
            locc = jnp.minimum(jnp.maximum(loc, 0), ROWS_PER_W - 1)
            plsc.store_scatter(tok_v, [locc], tok, mask=mask)
        return carry

    lax.fori_loop(0, S // (LANES * _SCAN_UNROLL), scan, 0)

    _pipelined_gather(x_hbm, tok_v, xe_hbm, base, ROWS_PER_W,
                      (b0, b1, b2), (g0, g1, g2), (w0, w1, w2))


def _dispatch_gather(slots_flat, xbf3):
    mesh = plsc.VectorSubcoreMesh(
        core_axis_name="c", subcore_axis_name="s",
        num_cores=NC, num_subcores=NS)
    return pl.kernel(
        _dispatch_body,
        out_type=jax.ShapeDtypeStruct((EC, XC), jnp.int32),
        mesh=mesh,
        compiler_params=pltpu.CompilerParams(needs_layout_passes=False),
        scratch_types=[
            pltpu.VMEM((S,), jnp.int32),
            pltpu.VMEM((ROWS_PER_W,), jnp.int32),
        ] + [pltpu.VMEM((RC, XC), jnp.int32)] * NBUF
          + [pltpu.SemaphoreType.DMA] * (2 * NBUF),
    )(slots_flat, xbf3)


BF1 = 512
BD2 = 512


def _ffn1_body(xe_ref, wg_ref, wu_ref, h_ref):
    xb = xe_ref[0]
    g = jnp.dot(xb, wg_ref[0].astype(jnp.bfloat16),
                preferred_element_type=jnp.float32)
    u = jnp.dot(xb, wu_ref[0].astype(jnp.bfloat16),
                preferred_element_type=jnp.float32)
    h_ref[0] = (g * jax.nn.sigmoid(g) * u).astype(jnp.bfloat16)


def _ffn1(xe3, w_gate, w_up):
    return pl.pallas_call(
        _ffn1_body,
        grid=(E, DFF // BF1),
        in_specs=[
            pl.BlockSpec((1, CAP, D), lambda e, f: (e, 0, 0)),
            pl.BlockSpec((1, D, BF1), lambda e, f: (e, 0, f)),
            pl.BlockSpec((1, D, BF1), lambda e, f: (e, 0, f)),
        ],
        out_specs=pl.BlockSpec((1, CAP, BF1), lambda e, f: (e, 0, f)),
        out_shape=jax.ShapeDtypeStruct((E, CAP, DFF), jnp.bfloat16),
    )(xe3, w_gate, w_up)


def _ffn2_body(h_ref, wd_ref, ye_ref):
    ye_ref[0] = jnp.dot(h_ref[0], wd_ref[0].astype(jnp.bfloat16),
                        preferred_element_type=jnp.float32
                        ).astype(jnp.bfloat16)


def _ffn2(h, w_down):
    return pl.pallas_call(
        _ffn2_body,
        grid=(E, D // BD2),
        in_specs=[
            pl.BlockSpec((1, CAP, DFF), lambda e, d: (e, 0, 0)),
            pl.BlockSpec((1, DFF, BD2), lambda e, d: (e, 0, d)),
        ],
        out_specs=pl.BlockSpec((1, CAP, BD2), lambda e, d: (e, 0, d)),
        out_shape=jax.ShapeDtypeStruct((E, CAP, D), jnp.bfloat16),
    )(h, w_down)


def _combine_body(ye_hbm, s0_hbm, s1_hbm, yg0_hbm, yg1_hbm,
                  idx0_v, idx1_v, b0, b1, b2, g0, g1, g2, w0, w1, w2):
    wid = lax.axis_index("s") * NC + lax.axis_index("c")
    tbase = wid * TOK_PER_W

    for iv, s_hbm, o_hbm in ((idx0_v, s0_hbm, yg0_hbm),
                             (idx1_v, s1_hbm, yg1_hbm)):
        pltpu.sync_copy(s_hbm.at[pl.ds(tbase, TOK_PER_W)], iv)
        for j in range(TOK_PER_W // LANES):
            sl = pl.ds(j * LANES, LANES)
            iv[sl] = jnp.minimum(iv[sl], EC - 1)
        _pipelined_gather(ye_hbm, iv, o_hbm, tbase, TOK_PER_W,
                          (b0, b1, b2), (g0, g1, g2), (w0, w1, w2))


def _combine_gather(ye, slots0, slots1):
    mesh = plsc.VectorSubcoreMesh(
        core_axis_name="c", subcore_axis_name="s",
        num_cores=NC, num_subcores=NS)
    return pl.kernel(
        _combine_body,
        out_type=[
            jax.ShapeDtypeStruct((T, XC), jnp.int32),
            jax.ShapeDtypeStruct((T, XC), jnp.int32),
        ],
        mesh=mesh,
        compiler_params=pltpu.CompilerParams(needs_layout_passes=False),
        scratch_types=[
            pltpu.VMEM((TOK_PER_W,), jnp.int32),
            pltpu.VMEM((TOK_PER_W,), jnp.int32),
        ] + [pltpu.VMEM((RC, XC), jnp.int32)] * NBUF
          + [pltpu.SemaphoreType.DMA] * (2 * NBUF),
    )(ye, slots0, slots1)


def _shared1_body(x_ref, wsg_ref, wsu_ref, hsh_ref):
    xb = x_ref[...].astype(jnp.bfloat16)
    g = jnp.dot(xb, wsg_ref[...].astype(jnp.bfloat16),
                preferred_element_type=jnp.float32)
    u = jnp.dot(xb, wsu_ref[...].astype(jnp.bfloat16),
                preferred_element_type=jnp.float32)
    hsh_ref[...] = (g * jax.nn.sigmoid(g) * u).astype(jnp.bfloat16)


def _shared1(x, ws_gate, ws_up):
    return pl.pallas_call(
        _shared1_body,
        grid=(T // BT,),
        in_specs=[
            pl.BlockSpec((BT, D), lambda i: (i, 0)),
            pl.BlockSpec((D, DSH), lambda i: (0, 0)),
            pl.BlockSpec((D, DSH), lambda i: (0, 0)),
        ],
        out_specs=pl.BlockSpec((BT, DSH), lambda i: (i, 0)),
        out_shape=jax.ShapeDtypeStruct((T, DSH), jnp.bfloat16),
    )(x, ws_gate, ws_up)


def _final_body(hsh_ref, wsd_ref, yg0_ref, yg1_ref, ws_ref, y_ref):
    ysh = jnp.dot(hsh_ref[...], wsd_ref[...].astype(jnp.bfloat16),
                  preferred_element_type=jnp.float32)
    w0 = ws_ref[:, 0:1]
    w1 = ws_ref[:, 1:2]
    y_ref[...] = (ysh + w0 * yg0_ref[...].astype(jnp.float32)
                  + w1 * yg1_ref[...].astype(jnp.float32))


def _final(hsh, ws_down, yg0, yg1, ws):
    return pl.pallas_call(
        _final_body,
        grid=(T // BT,),
        in_specs=[
            pl.BlockSpec((BT, DSH), lambda i: (i, 0)),
            pl.BlockSpec((DSH, D), lambda i: (0, 0)),
            pl.BlockSpec((BT, D), lambda i: (i, 0)),
            pl.BlockSpec((BT, D), lambda i: (i, 0)),
            pl.BlockSpec((BT, K), lambda i: (i, 0)),
        ],
        out_specs=pl.BlockSpec((BT, D), lambda i: (i, 0)),
        out_shape=jax.ShapeDtypeStruct((T, D), jnp.float32),
    )(hsh, ws_down, yg0, yg1, ws)


def _pack(a):
    return lax.bitcast_convert_type(
        a.reshape(*a.shape[:-1], a.shape[-1] // 2, 2), jnp.int32)


def _unpack(a):
    b = lax.bitcast_convert_type(a, jnp.bfloat16)
    return b.reshape(*a.shape[:-1], a.shape[-1] * 2)


def kernel(x, gate_w, w_gate, w_up, w_down, ws_gate, ws_up, ws_down):
    gwt = gate_w.T
    slots, ws, xbf = _router(x, gwt)
    xe = _dispatch_gather(slots.reshape(S), _pack(xbf))
    hsh = _shared1(x, ws_gate, ws_up)
    h = _ffn1(_unpack(xe).reshape(E, CAP, D), w_gate, w_up)
    ye = _ffn2(h, w_down)
    yg0, yg1 = _combine_gather(_pack(ye.reshape(EC, D)),
                               slots[:, 0], slots[:, 1])
    return _final(hsh, ws_down, _unpack(yg0), _unpack(yg1), ws)


# speedup vs baseline: 3.0482x; 3.0482x over previous
import functools

import jax
import jax.numpy as jnp
from jax import lax
from jax.experimental import pallas as pl
from jax.experimental.pallas import tpu as pltpu
from jax.experimental.pallas import tpu_sc as plsc

T = 2048
D = 2048
E = 8
K = 2
DFF = 1024
DSH = 1024
CAP = 640
S = T * K
EC = E * CAP

NC = 2
NS = 16
NW = NC * NS
LANES = 16

BT = 256
ROWS_PER_W = EC // NW
TOK_PER_W = T // NW
RC = 32
XC = D // 2


def _pack_pairs(bf):
    n = bf.shape[-1] // 2
    b = lax.bitcast_convert_type(bf, jnp.uint16)
    lo = b[:, :n].astype(jnp.uint32)
    hi = b[:, n:].astype(jnp.uint32)
    return lax.bitcast_convert_type(lo | (hi << 16), jnp.int32)


def _unpack_pairs(pi):
    u = lax.bitcast_convert_type(pi, jnp.uint32)
    lo = lax.convert_element_type(u & jnp.uint32(0xFFFF), jnp.uint16)
    hi = lax.convert_element_type(u >> 16, jnp.uint16)
    return jnp.concatenate(
        [lax.bitcast_convert_type(lo, jnp.bfloat16),
         lax.bitcast_convert_type(hi, jnp.bfloat16)], axis=-1)
NBUF = 3


def _pipelined_gather(src_hbm, idx_v, dst_hbm, dst_base, nrows,
                      bufs, gsems, wsems):
    nb = len(bufs)
    nch = nrows // RC

    def g(cc):
        return pltpu.async_copy(
            src_hbm.at[idx_v.at[pl.ds(cc * RC, RC)]], bufs[cc % nb],
            gsems[cc % nb])

    gd = {}
    wd = {}
    for b in range(min(nb, nch)):
        gd[b] = g(b)
    for cc in range(nch):
        gd[cc].wait()
        wd[cc] = pltpu.async_copy(
            bufs[cc % nb], dst_hbm.at[pl.ds(dst_base + cc * RC, RC)],
            wsems[cc % nb])
        if cc + nb < nch:
            wd[cc].wait()
            gd[cc + nb] = g(cc + nb)
    for cc in range(max(0, nch - nb), nch):
        wd[cc].wait()


def _router_body(x_ref, gwt_ref, slots_ref, ws_ref, xpack_ref, carry_ref):
    i = pl.program_id(0)

    @pl.when(i == 0)
    def _():
        carry_ref[...] = jnp.zeros_like(carry_ref)

    xb = x_ref[...]
    logits = jnp.dot(xb, gwt_ref[...], preferred_element_type=jnp.float32)
    scores = jax.nn.sigmoid(logits)
    lane = lax.broadcasted_iota(jnp.int32, (BT, E), 1)

    m1 = jnp.max(scores, axis=1, keepdims=True)
    e1 = jnp.min(jnp.where(scores == m1, lane, E), axis=1, keepdims=True)
    masked = jnp.where(lane == e1, -1.0, scores)
    m2 = jnp.max(masked, axis=1, keepdims=True)
    e2 = jnp.min(jnp.where(masked == m2, lane, E), axis=1, keepdims=True)

    wsum = m1 + m2 + 1e-20
    w1 = m1 / wsum
    w2 = m2 / wsum

    oh1 = (lane == e1).astype(jnp.float32)
    oh2 = (lane == e2).astype(jnp.float32)
    ohsum = oh1 + oh2

    r = lax.broadcasted_iota(jnp.int32, (BT, BT), 0)
    c = lax.broadcasted_iota(jnp.int32, (BT, BT), 1)
    tril = (c < r).astype(jnp.float32)
    cnt = jnp.dot(tril, ohsum, preferred_element_type=jnp.float32)
    cnt = cnt + carry_ref[...]
    carry_ref[...] = carry_ref[...] + jnp.sum(ohsum, axis=0, keepdims=True)

    pos1 = jnp.sum(jnp.where(oh1 > 0, cnt, 0.0), axis=1, keepdims=True)
    pos2 = jnp.sum(jnp.where(oh2 > 0, cnt, 0.0), axis=1, keepdims=True)
    p1 = pos1.astype(jnp.int32)
    p2 = pos2.astype(jnp.int32)
    v1 = p1 < CAP
    v2 = p2 < CAP
    slot1 = jnp.where(v1, e1 * CAP + p1, EC)
    slot2 = jnp.where(v2, e2 * CAP + p2, EC)
    slots_ref[...] = jnp.concatenate([slot1, slot2], axis=1)
    ws_ref[...] = jnp.concatenate(
        [jnp.where(v1, w1, 0.0), jnp.where(v2, w2, 0.0)], axis=1)
    xpack_ref[...] = _pack_pairs(xb.astype(jnp.bfloat16))


def _router(x, gwt):
    return pl.pallas_call(
        _router_body,
        grid=(T // BT,),
        in_specs=[
            pl.BlockSpec((BT, D), lambda i: (i, 0)),
            pl.BlockSpec((D, E), lambda i: (0, 0)),
        ],
        out_specs=[
            pl.BlockSpec((BT, K), lambda i: (i, 0)),
            pl.BlockSpec((BT, K), lambda i: (i, 0)),
            pl.BlockSpec((BT, XC), lambda i: (i, 0)),
        ],
        out_shape=[
            jax.ShapeDtypeStruct((T, K), jnp.int32),
            jax.ShapeDtypeStruct((T, K), jnp.float32),
            jax.ShapeDtypeStruct((T, XC), jnp.int32),
        ],
        scratch_shapes=[pltpu.VMEM((1, E), jnp.float32)],
    )(x, gwt)


_SCAN_UNROLL = 4


def _dispatch_body(slots_hbm, x_hbm, xe_hbm, slots_v, tok_v,
                   b0, b1, b2, g0, g1, g2, w0, w1, w2):
    wid = lax.axis_index("s") * NC + lax.axis_index("c")
    base = wid * ROWS_PER_W

    pltpu.sync_copy(slots_hbm, slots_v)
    for j in range(ROWS_PER_W // LANES):
        tok_v[pl.ds(j * LANES, LANES)] = jnp.zeros((LANES,), jnp.int32)

    iot = lax.iota(jnp.int32, LANES)

    def scan(j, carry):
        for u in range(_SCAN_UNROLL):
            off = j * (LANES * _SCAN_UNROLL) + u * LANES
            idx = slots_v[pl.ds(off, LANES)]
            tok = jnp.right_shift(off + iot, 1)
            loc = idx - base
            mask = (loc >= 0) & (loc < ROWS_PER_W)
            locc = jnp.minimum(jnp.maximum(loc, 0), ROWS_PER_W - 1)
            plsc.store_scatter(tok_v, [locc], tok, mask=mask)
        return carry

    lax.fori_loop(0, S // (LANES * _SCAN_UNROLL), scan, 0)

    _pipelined_gather(x_hbm, tok_v, xe_hbm, base, ROWS_PER_W,
                      (b0, b1, b2), (g0, g1, g2), (w0, w1, w2))


def _dispatch_gather(slots_flat, xbf3):
    mesh = plsc.VectorSubcoreMesh(
        core_axis_name="c", subcore_axis_name="s",
        num_cores=NC, num_subcores=NS)
    return pl.kernel(
        _dispatch_body,
        out_type=jax.ShapeDtypeStruct((EC, XC), jnp.int32),
        mesh=mesh,
        compiler_params=pltpu.CompilerParams(needs_layout_passes=False),
        scratch_types=[
            pltpu.VMEM((S,), jnp.int32),
            pltpu.VMEM((ROWS_PER_W,), jnp.int32),
        ] + [pltpu.VMEM((RC, XC), jnp.int32)] * NBUF
          + [pltpu.SemaphoreType.DMA] * (2 * NBUF),
    )(slots_flat, xbf3)


BF1 = 512
BD2 = 512


def _ffn1_body(xe_ref, wg_ref, wu_ref, h_ref):
    xb = _unpack_pairs(xe_ref[0])
    g = jnp.dot(xb, wg_ref[0].astype(jnp.bfloat16),
                preferred_element_type=jnp.float32)
    u = jnp.dot(xb, wu_ref[0].astype(jnp.bfloat16),
                preferred_element_type=jnp.float32)
    h_ref[0] = (g * jax.nn.sigmoid(g) * u).astype(jnp.bfloat16)


def _ffn1(xe3, w_gate, w_up):
    return pl.pallas_call(
        _ffn1_body,
        grid=(E, DFF // BF1),
        in_specs=[
            pl.BlockSpec((1, CAP, XC), lambda e, f: (e, 0, 0)),
            pl.BlockSpec((1, D, BF1), lambda e, f: (e, 0, f)),
            pl.BlockSpec((1, D, BF1), lambda e, f: (e, 0, f)),
        ],
        out_specs=pl.BlockSpec((1, CAP, BF1), lambda e, f: (e, 0, f)),
        out_shape=jax.ShapeDtypeStruct((E, CAP, DFF), jnp.bfloat16),
    )(xe3, w_gate, w_up)


def _ffn2_body(h_ref, wd_ref, ye_ref):
    yb = jnp.dot(h_ref[0], wd_ref[0].astype(jnp.bfloat16),
                 preferred_element_type=jnp.float32).astype(jnp.bfloat16)
    ye_ref[0] = _pack_pairs(yb)


def _ffn2(h, w_down):
    return pl.pallas_call(
        _ffn2_body,
        grid=(E, D // BD2),
        in_specs=[
            pl.BlockSpec((1, CAP, DFF), lambda e, d: (e, 0, 0)),
            pl.BlockSpec((1, DFF, BD2), lambda e, d: (e, 0, d)),
        ],
        out_specs=pl.BlockSpec((1, CAP, BD2 // 2), lambda e, d: (e, 0, d)),
        out_shape=jax.ShapeDtypeStruct((E, CAP, XC), jnp.int32),
    )(h, w_down)


def _combine_body(ye_hbm, s0_hbm, s1_hbm, yg0_hbm, yg1_hbm,
                  idx0_v, idx1_v, b0, b1, b2, g0, g1, g2, w0, w1, w2):
    wid = lax.axis_index("s") * NC + lax.axis_index("c")
    tbase = wid * TOK_PER_W

    for iv, s_hbm, o_hbm in ((idx0_v, s0_hbm, yg0_hbm),
                             (idx1_v, s1_hbm, yg1_hbm)):
        pltpu.sync_copy(s_hbm.at[pl.ds(tbase, TOK_PER_W)], iv)
        for j in range(TOK_PER_W // LANES):
            sl = pl.ds(j * LANES, LANES)
            iv[sl] = jnp.minimum(iv[sl], EC - 1)
        _pipelined_gather(ye_hbm, iv, o_hbm, tbase, TOK_PER_W,
                          (b0, b1, b2), (g0, g1, g2), (w0, w1, w2))


def _combine_gather(ye, slots0, slots1):
    mesh = plsc.VectorSubcoreMesh(
        core_axis_name="c", subcore_axis_name="s",
        num_cores=NC, num_subcores=NS)
    return pl.kernel(
        _combine_body,
        out_type=[
            jax.ShapeDtypeStruct((T, XC), jnp.int32),
            jax.ShapeDtypeStruct((T, XC), jnp.int32),
        ],
        mesh=mesh,
        compiler_params=pltpu.CompilerParams(needs_layout_passes=False),
        scratch_types=[
            pltpu.VMEM((TOK_PER_W,), jnp.int32),
            pltpu.VMEM((TOK_PER_W,), jnp.int32),
        ] + [pltpu.VMEM((RC, XC), jnp.int32)] * NBUF
          + [pltpu.SemaphoreType.DMA] * (2 * NBUF),
    )(ye, slots0, slots1)


def _shared1_body(x_ref, wsg_ref, wsu_ref, hsh_ref):
    xb = x_ref[...].astype(jnp.bfloat16)
    g = jnp.dot(xb, wsg_ref[...].astype(jnp.bfloat16),
                preferred_element_type=jnp.float32)
    u = jnp.dot(xb, wsu_ref[...].astype(jnp.bfloat16),
                preferred_element_type=jnp.float32)
    hsh_ref[...] = (g * jax.nn.sigmoid(g) * u).astype(jnp.bfloat16)


def _shared1(x, ws_gate, ws_up):
    return pl.pallas_call(
        _shared1_body,
        grid=(T // BT,),
        in_specs=[
            pl.BlockSpec((BT, D), lambda i: (i, 0)),
            pl.BlockSpec((D, DSH), lambda i: (0, 0)),
            pl.BlockSpec((D, DSH), lambda i: (0, 0)),
        ],
        out_specs=pl.BlockSpec((BT, DSH), lambda i: (i, 0)),
        out_shape=jax.ShapeDtypeStruct((T, DSH), jnp.bfloat16),
    )(x, ws_gate, ws_up)


def _unpack_ye_row(pi):
    n = BD2 // 2
    return jnp.concatenate(
        [_unpack_pairs(pi[:, d * n:(d + 1) * n]) for d in range(D // BD2)],
        axis=-1)


def _final_body(hsh_ref, wsd_ref, yg0_ref, yg1_ref, ws_ref, y_ref):
    ysh = jnp.dot(hsh_ref[...], wsd_ref[...].astype(jnp.bfloat16),
                  preferred_element_type=jnp.float32)
    w0 = ws_ref[:, 0:1]
    w1 = ws_ref[:, 1:2]
    y_ref[...] = (ysh + w0 * _unpack_ye_row(yg0_ref[...]).astype(jnp.float32)
                  + w1 * _unpack_ye_row(yg1_ref[...]).astype(jnp.float32))


def _final(hsh, ws_down, yg0, yg1, ws):
    return pl.pallas_call(
        _final_body,
        grid=(T // BT,),
        in_specs=[
            pl.BlockSpec((BT, DSH), lambda i: (i, 0)),
            pl.BlockSpec((DSH, D), lambda i: (0, 0)),
            pl.BlockSpec((BT, XC), lambda i: (i, 0)),
            pl.BlockSpec((BT, XC), lambda i: (i, 0)),
            pl.BlockSpec((BT, K), lambda i: (i, 0)),
        ],
        out_specs=pl.BlockSpec((BT, D), lambda i: (i, 0)),
        out_shape=jax.ShapeDtypeStruct((T, D), jnp.float32),
    )(hsh, ws_down, yg0, yg1, ws)


def kernel(x, gate_w, w_gate, w_up, w_down, ws_gate, ws_up, ws_down):
    gwt = gate_w.T
    slots, ws, xpack = _router(x, gwt)
    xe = _dispatch_gather(slots.reshape(S), xpack)
    hsh = _shared1(x, ws_gate, ws_up)
    h = _ffn1(xe.reshape(E, CAP, XC), w_gate, w_up)
    ye = _ffn2(h, w_down)
    yg0, yg1 = _combine_gather(ye.reshape(EC, XC),
                               slots[:, 0], slots[:, 1])
    return _final(hsh, ws_down, yg0, yg1, ws)


# speedup vs baseline: 3.9631x; 1.3002x over previous
import functools

import jax
import jax.numpy as jnp
from jax import lax
from jax.experimental import pallas as pl
from jax.experimental.pallas import tpu as pltpu
from jax.experimental.pallas import tpu_sc as plsc

T = 2048
D = 2048
E = 8
K = 2
DFF = 1024
DSH = 1024
CAP = 640
S = T * K
EC = E * CAP

NC = 2
NS = 16
NW = NC * NS
LANES = 16

BT = 256
ROWS_PER_W = EC // NW
TOK_PER_W = T // NW
RC = 32
XC = D // 2


def _pack_pairs(bf):
    n = bf.shape[-1] // 2
    b = lax.bitcast_convert_type(bf, jnp.uint16)
    lo = b[:, :n].astype(jnp.uint32)
    hi = b[:, n:].astype(jnp.uint32)
    return lax.bitcast_convert_type(lo | (hi << 16), jnp.int32)


def _unpack_pairs(pi):
    u = lax.bitcast_convert_type(pi, jnp.uint32)
    lo = lax.convert_element_type(u & jnp.uint32(0xFFFF), jnp.uint16)
    hi = lax.convert_element_type(u >> 16, jnp.uint16)
    return jnp.concatenate(
        [lax.bitcast_convert_type(lo, jnp.bfloat16),
         lax.bitcast_convert_type(hi, jnp.bfloat16)], axis=-1)
NBUF = 3


def _pipelined_gather(src_hbm, idx_v, dst_hbm, dst_base, nrows,
                      bufs, gsems, wsems, cond=None):
    nb = len(bufs)
    nch = nrows // RC

    def run(cc, fn):
        if cond is None:
            fn()
        else:
            pl.when(cond(cc))(fn)

    def gsrc(cc):
        return src_hbm.at[idx_v.at[pl.ds(cc * RC, RC)]]

    def wdst(cc):
        return dst_hbm.at[pl.ds(dst_base + cc * RC, RC)]

    def g_issue(cc):
        def fn():
            pltpu.async_copy(gsrc(cc), bufs[cc % nb], gsems[cc % nb])
        run(cc, fn)

    def g_wait(cc):
        def fn():
            pltpu.make_async_copy(gsrc(cc), bufs[cc % nb],
                                  gsems[cc % nb]).wait()
        run(cc, fn)

    def w_issue(cc):
        def fn():
            pltpu.async_copy(bufs[cc % nb], wdst(cc), wsems[cc % nb])
        run(cc, fn)

    def w_wait(cc):
        def fn():
            pltpu.make_async_copy(bufs[cc % nb], wdst(cc),
                                  wsems[cc % nb]).wait()
        run(cc, fn)

    for b in range(min(nb, nch)):
        g_issue(b)
    for cc in range(nch):
        g_wait(cc)
        w_issue(cc)
        if cc + nb < nch:
            w_wait(cc)
            g_issue(cc + nb)
    for cc in range(max(0, nch - nb), nch):
        w_wait(cc)


def _router_body(x_ref, gwt_ref, slots_ref, slotsc_ref, ws_ref, xpack_ref,
                 counts_ref, carry_ref):
    i = pl.program_id(0)

    @pl.when(i == 0)
    def _():
        carry_ref[...] = jnp.zeros_like(carry_ref)

    xb = x_ref[...]
    logits = jnp.dot(xb, gwt_ref[...], preferred_element_type=jnp.float32)
    scores = jax.nn.sigmoid(logits)
    lane = lax.broadcasted_iota(jnp.int32, (BT, E), 1)

    m1 = jnp.max(scores, axis=1, keepdims=True)
    e1 = jnp.min(jnp.where(scores == m1, lane, E), axis=1, keepdims=True)
    masked = jnp.where(lane == e1, -1.0, scores)
    m2 = jnp.max(masked, axis=1, keepdims=True)
    e2 = jnp.min(jnp.where(masked == m2, lane, E), axis=1, keepdims=True)

    wsum = m1 + m2 + 1e-20
    w1 = m1 / wsum
    w2 = m2 / wsum

    oh1 = (lane == e1).astype(jnp.float32)
    oh2 = (lane == e2).astype(jnp.float32)
    ohsum = oh1 + oh2

    r = lax.broadcasted_iota(jnp.int32, (BT, BT), 0)
    c = lax.broadcasted_iota(jnp.int32, (BT, BT), 1)
    tril = (c < r).astype(jnp.float32)
    cnt = jnp.dot(tril, ohsum, preferred_element_type=jnp.float32)
    cnt = cnt + carry_ref[...]
    carry_ref[...] = carry_ref[...] + jnp.sum(ohsum, axis=0, keepdims=True)
    counts_ref[...] = jnp.concatenate(
        [carry_ref[...], jnp.zeros_like(carry_ref[...])], axis=1
    ).astype(jnp.int32)

    pos1 = jnp.sum(jnp.where(oh1 > 0, cnt, 0.0), axis=1, keepdims=True)
    pos2 = jnp.sum(jnp.where(oh2 > 0, cnt, 0.0), axis=1, keepdims=True)
    p1 = pos1.astype(jnp.int32)
    p2 = pos2.astype(jnp.int32)
    v1 = p1 < CAP
    v2 = p2 < CAP
    slot1 = e1 * CAP + jnp.minimum(p1, CAP - 1)
    slot2 = e2 * CAP + jnp.minimum(p2, CAP - 1)
    slots_ref[...] = jnp.concatenate(
        [jnp.where(v1, slot1, EC), jnp.where(v2, slot2, EC)], axis=1)
    slotsc_ref[...] = jnp.concatenate([slot1, slot2], axis=1)
    ws_ref[...] = jnp.concatenate(
        [jnp.where(v1, w1, 0.0), jnp.where(v2, w2, 0.0)], axis=1)
    xpack_ref[...] = _pack_pairs(xb.astype(jnp.bfloat16))


def _router(x, gwt):
    return pl.pallas_call(
        _router_body,
        grid=(T // BT,),
        in_specs=[
            pl.BlockSpec((BT, D), lambda i: (i, 0)),
            pl.BlockSpec((D, E), lambda i: (0, 0)),
        ],
        out_specs=[
            pl.BlockSpec((BT, K), lambda i: (i, 0)),
            pl.BlockSpec((BT, K), lambda i: (i, 0)),
            pl.BlockSpec((BT, K), lambda i: (i, 0)),
            pl.BlockSpec((BT, XC), lambda i: (i, 0)),
            pl.BlockSpec((1, 2 * E), lambda i: (0, 0)),
        ],
        out_shape=[
            jax.ShapeDtypeStruct((T, K), jnp.int32),
            jax.ShapeDtypeStruct((T, K), jnp.int32),
            jax.ShapeDtypeStruct((T, K), jnp.float32),
            jax.ShapeDtypeStruct((T, XC), jnp.int32),
            jax.ShapeDtypeStruct((1, 2 * E), jnp.int32),
        ],
        scratch_shapes=[pltpu.VMEM((1, E), jnp.float32)],
    )(x, gwt)


_SCAN_UNROLL = 4


def _dispatch_body(s0_hbm, s1_hbm, counts_hbm, x_hbm, xe_hbm,
                   slots_v, tok_v, cnt_v,
                   b0, b1, b2, g0, g1, g2, w0, w1, w2):
    wid = lax.axis_index("s") * NC + lax.axis_index("c")
    base = wid * ROWS_PER_W

    pltpu.sync_copy(counts_hbm.at[0], cnt_v)
    iot = lax.iota(jnp.int32, LANES)
    for j in range(ROWS_PER_W // LANES):
        tok_v[pl.ds(j * LANES, LANES)] = jnp.zeros((LANES,), jnp.int32)

    for s_hbm in (s0_hbm, s1_hbm):
        pltpu.sync_copy(s_hbm, slots_v)

        def scan(j, carry):
            for u in range(_SCAN_UNROLL):
                off = j * (LANES * _SCAN_UNROLL) + u * LANES
                idx = slots_v[pl.ds(off, LANES)]
                tok = off + iot
                loc = idx - base
                mask = (loc >= 0) & (loc < ROWS_PER_W)
                locc = jnp.minimum(jnp.maximum(loc, 0), ROWS_PER_W - 1)
                plsc.store_scatter(tok_v, [locc], tok, mask=mask)
            return carry

        lax.fori_loop(0, T // (LANES * _SCAN_UNROLL), scan, 0)

    cnts = cnt_v[pl.ds(0, LANES)]
    my_e = wid // (CAP // ROWS_PER_W)
    my_cnt = jnp.sum(jnp.where(lax.iota(jnp.int32, LANES) == my_e, cnts, 0))
    nvalid = my_cnt - (wid % (CAP // ROWS_PER_W)) * ROWS_PER_W
    _pipelined_gather(x_hbm, tok_v, xe_hbm, base, ROWS_PER_W,
                      (b0, b1, b2), (g0, g1, g2), (w0, w1, w2),
                      cond=lambda cc: cc * RC < nvalid)


def _dispatch_gather(slots0, slots1, counts, xpack):
    mesh = plsc.VectorSubcoreMesh(
        core_axis_name="c", subcore_axis_name="s",
        num_cores=NC, num_subcores=NS)
    return pl.kernel(
        _dispatch_body,
        out_type=jax.ShapeDtypeStruct((EC, XC), jnp.int32),
        mesh=mesh,
        compiler_params=pltpu.CompilerParams(needs_layout_passes=False),
        scratch_types=[
            pltpu.VMEM((T,), jnp.int32),
            pltpu.VMEM((ROWS_PER_W,), jnp.int32),
            pltpu.VMEM((2 * E,), jnp.int32),
        ] + [pltpu.VMEM((RC, XC), jnp.int32)] * NBUF
          + [pltpu.SemaphoreType.DMA] * (2 * NBUF),
    )(slots0, slots1, counts, xpack)


BF1 = 1024
BD2 = 1024


def _ffn1_body(xe_ref, wg_ref, wu_ref, h_ref):
    xb = _unpack_pairs(xe_ref[0])
    g = jnp.dot(xb, wg_ref[0].astype(jnp.bfloat16),
                preferred_element_type=jnp.float32)
    u = jnp.dot(xb, wu_ref[0].astype(jnp.bfloat16),
                preferred_element_type=jnp.float32)
    h_ref[0] = (g * jax.nn.sigmoid(g) * u).astype(jnp.bfloat16)


def _ffn1(xe3, w_gate, w_up):
    return pl.pallas_call(
        _ffn1_body,
        grid=(E, DFF // BF1),
        in_specs=[
            pl.BlockSpec((1, CAP, XC), lambda e, f: (e, 0, 0)),
            pl.BlockSpec((1, D, BF1), lambda e, f: (e, 0, f)),
            pl.BlockSpec((1, D, BF1), lambda e, f: (e, 0, f)),
        ],
        out_specs=pl.BlockSpec((1, CAP, BF1), lambda e, f: (e, 0, f)),
        out_shape=jax.ShapeDtypeStruct((E, CAP, DFF), jnp.bfloat16),
    )(xe3, w_gate, w_up)


def _ffn2_body(h_ref, wd_ref, ye_ref):
    yb = jnp.dot(h_ref[0], wd_ref[0].astype(jnp.bfloat16),
                 preferred_element_type=jnp.float32).astype(jnp.bfloat16)
    ye_ref[0] = _pack_pairs(yb)


def _ffn2(h, w_down):
    return pl.pallas_call(
        _ffn2_body,
        grid=(E, D // BD2),
        in_specs=[
            pl.BlockSpec((1, CAP, DFF), lambda e, d: (e, 0, 0)),
            pl.BlockSpec((1, DFF, BD2), lambda e, d: (e, 0, d)),
        ],
        out_specs=pl.BlockSpec((1, CAP, BD2 // 2), lambda e, d: (e, 0, d)),
        out_shape=jax.ShapeDtypeStruct((E, CAP, XC), jnp.int32),
    )(h, w_down)


def _combine_body(ye_hbm, s0_hbm, s1_hbm, yg0_hbm, yg1_hbm,
                  idx0_v, idx1_v, b0, b1, b2, g0, g1, g2, w0, w1, w2):
    wid = lax.axis_index("s") * NC + lax.axis_index("c")
    tbase = wid * TOK_PER_W

    for iv, s_hbm, o_hbm in ((idx0_v, s0_hbm, yg0_hbm),
                             (idx1_v, s1_hbm, yg1_hbm)):
        pltpu.sync_copy(s_hbm.at[pl.ds(tbase, TOK_PER_W)], iv)
        _pipelined_gather(ye_hbm, iv, o_hbm, tbase, TOK_PER_W,
                          (b0, b1, b2), (g0, g1, g2), (w0, w1, w2))


def _combine_gather(ye, slots0, slots1):
    mesh = plsc.VectorSubcoreMesh(
        core_axis_name="c", subcore_axis_name="s",
        num_cores=NC, num_subcores=NS)
    return pl.kernel(
        _combine_body,
        out_type=[
            jax.ShapeDtypeStruct((T, XC), jnp.int32),
            jax.ShapeDtypeStruct((T, XC), jnp.int32),
        ],
        mesh=mesh,
        compiler_params=pltpu.CompilerParams(needs_layout_passes=False),
        scratch_types=[
            pltpu.VMEM((TOK_PER_W,), jnp.int32),
            pltpu.VMEM((TOK_PER_W,), jnp.int32),
        ] + [pltpu.VMEM((RC, XC), jnp.int32)] * NBUF
          + [pltpu.SemaphoreType.DMA] * (2 * NBUF),
    )(ye, slots0, slots1)


def _shared1_body(x_ref, wsg_ref, wsu_ref, hsh_ref):
    xb = x_ref[...].astype(jnp.bfloat16)
    g = jnp.dot(xb, wsg_ref[...].astype(jnp.bfloat16),
                preferred_element_type=jnp.float32)
    u = jnp.dot(xb, wsu_ref[...].astype(jnp.bfloat16),
                preferred_element_type=jnp.float32)
    hsh_ref[...] = (g * jax.nn.sigmoid(g) * u).astype(jnp.bfloat16)


def _shared1(x, ws_gate, ws_up):
    return pl.pallas_call(
        _shared1_body,
        grid=(T // BT,),
        in_specs=[
            pl.BlockSpec((BT, D), lambda i: (i, 0)),
            pl.BlockSpec((D, DSH), lambda i: (0, 0)),
            pl.BlockSpec((D, DSH), lambda i: (0, 0)),
        ],
        out_specs=pl.BlockSpec((BT, DSH), lambda i: (i, 0)),
        out_shape=jax.ShapeDtypeStruct((T, DSH), jnp.bfloat16),
    )(x, ws_gate, ws_up)


def _unpack_ye_row(pi):
    n = BD2 // 2
    return jnp.concatenate(
        [_unpack_pairs(pi[:, d * n:(d + 1) * n]) for d in range(D // BD2)],
        axis=-1)


def _final_body(hsh_ref, wsd_ref, yg0_ref, yg1_ref, ws_ref, y_ref):
    ysh = jnp.dot(hsh_ref[...], wsd_ref[...].astype(jnp.bfloat16),
                  preferred_element_type=jnp.float32)
    w0 = ws_ref[:, 0:1]
    w1 = ws_ref[:, 1:2]
    y_ref[...] = (ysh + w0 * _unpack_ye_row(yg0_ref[...]).astype(jnp.float32)
                  + w1 * _unpack_ye_row(yg1_ref[...]).astype(jnp.float32))


def _final(hsh, ws_down, yg0, yg1, ws):
    return pl.pallas_call(
        _final_body,
        grid=(T // BT,),
        in_specs=[
            pl.BlockSpec((BT, DSH), lambda i: (i, 0)),
            pl.BlockSpec((DSH, D), lambda i: (0, 0)),
            pl.BlockSpec((BT, XC), lambda i: (i, 0)),
            pl.BlockSpec((BT, XC), lambda i: (i, 0)),
            pl.BlockSpec((BT, K), lambda i: (i, 0)),
        ],
        out_specs=pl.BlockSpec((BT, D), lambda i: (i, 0)),
        out_shape=jax.ShapeDtypeStruct((T, D), jnp.float32),
    )(hsh, ws_down, yg0, yg1, ws)


def kernel(x, gate_w, w_gate, w_up, w_down, ws_gate, ws_up, ws_down):
    gwt = gate_w.T
    slots, slotsc, ws, xpack, counts = _router(x, gwt)
    xe = _dispatch_gather(slots[:, 0], slots[:, 1], counts, xpack)
    hsh = _shared1(x, ws_gate, ws_up)
    h = _ffn1(xe.reshape(E, CAP, XC), w_gate, w_up)
    ye = _ffn2(h, w_down)
    yg0, yg1 = _combine_gather(ye.reshape(EC, XC),
                               slotsc[:, 0], slotsc[:, 1])
    return _final(hsh, ws_down, yg0, yg1, ws)


# speedup vs baseline: 4.0086x; 1.0115x over previous
import functools

import jax
import jax.numpy as jnp
from jax import lax
from jax.experimental import pallas as pl
from jax.experimental.pallas import tpu as pltpu
from jax.experimental.pallas import tpu_sc as plsc

T = 2048
D = 2048
E = 8
K = 2
DFF = 1024
DSH = 1024
CAP = 640
S = T * K
EC = E * CAP

NC = 2
NS = 16
NW = NC * NS
LANES = 16

BT = 256
ROWS_PER_W = EC // NW
TOK_PER_W = T // NW
RC = 32
XC = D // 2


def _pack_pairs(bf):
    n = bf.shape[-1] // 2
    b = lax.bitcast_convert_type(bf, jnp.uint16)
    lo = b[:, :n].astype(jnp.uint32)
    hi = b[:, n:].astype(jnp.uint32)
    return lax.bitcast_convert_type(lo | (hi << 16), jnp.int32)


def _unpack_pairs(pi):
    u = lax.bitcast_convert_type(pi, jnp.uint32)
    lo = lax.convert_element_type(u & jnp.uint32(0xFFFF), jnp.uint16)
    hi = lax.convert_element_type(u >> 16, jnp.uint16)
    return jnp.concatenate(
        [lax.bitcast_convert_type(lo, jnp.bfloat16),
         lax.bitcast_convert_type(hi, jnp.bfloat16)], axis=-1)
NBUF = 3


def _pipelined_gather(src_hbm, idx_v, dst_hbm, dst_base, nrows,
                      bufs, gsems, wsems, cond=None):
    nb = len(bufs)
    nch = nrows // RC

    def run(cc, fn):
        if cond is None:
            fn()
        else:
            pl.when(cond(cc))(fn)

    def gsrc(cc):
        return src_hbm.at[idx_v.at[pl.ds(cc * RC, RC)]]

    def wdst(cc):
        return dst_hbm.at[pl.ds(dst_base + cc * RC, RC)]

    def g_issue(cc):
        def fn():
            pltpu.async_copy(gsrc(cc), bufs[cc % nb], gsems[cc % nb])
        run(cc, fn)

    def g_wait(cc):
        def fn():
            pltpu.make_async_copy(gsrc(cc), bufs[cc % nb],
                                  gsems[cc % nb]).wait()
        run(cc, fn)

    def w_issue(cc):
        def fn():
            pltpu.async_copy(bufs[cc % nb], wdst(cc), wsems[cc % nb])
        run(cc, fn)

    def w_wait(cc):
        def fn():
            pltpu.make_async_copy(bufs[cc % nb], wdst(cc),
                                  wsems[cc % nb]).wait()
        run(cc, fn)

    for b in range(min(nb, nch)):
        g_issue(b)
    for cc in range(nch):
        g_wait(cc)
        w_issue(cc)
        if cc + nb < nch:
            w_wait(cc)
            g_issue(cc + nb)
    for cc in range(max(0, nch - nb), nch):
        w_wait(cc)


def _router_body(x_ref, gw_ref, slots_ref, slotsc_ref, ws_ref, xpack_ref,
                 counts_ref, carry_ref):
    i = pl.program_id(0)

    @pl.when(i == 0)
    def _():
        carry_ref[...] = jnp.zeros_like(carry_ref)

    xb = x_ref[...]
    logits = lax.dot_general(
        xb, gw_ref[...], (((1,), (1,)), ((), ())),
        preferred_element_type=jnp.float32)
    scores = jax.nn.sigmoid(logits)
    lane = lax.broadcasted_iota(jnp.int32, (BT, E), 1)

    m1 = jnp.max(scores, axis=1, keepdims=True)
    e1 = jnp.min(jnp.where(scores == m1, lane, E), axis=1, keepdims=True)
    masked = jnp.where(lane == e1, -1.0, scores)
    m2 = jnp.max(masked, axis=1, keepdims=True)
    e2 = jnp.min(jnp.where(masked == m2, lane, E), axis=1, keepdims=True)

    wsum = m1 + m2 + 1e-20
    w1 = m1 / wsum
    w2 = m2 / wsum

    oh1 = (lane == e1).astype(jnp.float32)
    oh2 = (lane == e2).astype(jnp.float32)
    ohsum = oh1 + oh2

    r = lax.broadcasted_iota(jnp.int32, (BT, BT), 0)
    c = lax.broadcasted_iota(jnp.int32, (BT, BT), 1)
    tril = (c < r).astype(jnp.float32)
    cnt = jnp.dot(tril, ohsum, preferred_element_type=jnp.float32)
    cnt = cnt + carry_ref[...]
    carry_ref[...] = carry_ref[...] + jnp.sum(ohsum, axis=0, keepdims=True)
    counts_ref[...] = jnp.concatenate(
        [carry_ref[...], jnp.zeros_like(carry_ref[...])], axis=1
    ).astype(jnp.int32)

    pos1 = jnp.sum(jnp.where(oh1 > 0, cnt, 0.0), axis=1, keepdims=True)
    pos2 = jnp.sum(jnp.where(oh2 > 0, cnt, 0.0), axis=1, keepdims=True)
    p1 = pos1.astype(jnp.int32)
    p2 = pos2.astype(jnp.int32)
    v1 = p1 < CAP
    v2 = p2 < CAP
    slot1 = e1 * CAP + jnp.minimum(p1, CAP - 1)
    slot2 = e2 * CAP + jnp.minimum(p2, CAP - 1)
    slots_ref[...] = jnp.concatenate(
        [jnp.where(v1, slot1, EC), jnp.where(v2, slot2, EC)], axis=1)
    slotsc_ref[...] = jnp.concatenate([slot1, slot2], axis=1)
    ws_ref[...] = jnp.concatenate(
        [jnp.where(v1, w1, 0.0), jnp.where(v2, w2, 0.0)], axis=1)
    xpack_ref[...] = _pack_pairs(xb.astype(jnp.bfloat16))


def _router(x, gw):
    return pl.pallas_call(
        _router_body,
        grid=(T // BT,),
        in_specs=[
            pl.BlockSpec((BT, D), lambda i: (i, 0)),
            pl.BlockSpec((E, D), lambda i: (0, 0)),
        ],
        out_specs=[
            pl.BlockSpec((BT, K), lambda i: (i, 0)),
            pl.BlockSpec((BT, K), lambda i: (i, 0)),
            pl.BlockSpec((BT, K), lambda i: (i, 0)),
            pl.BlockSpec((BT, XC), lambda i: (i, 0)),
            pl.BlockSpec((1, 2 * E), lambda i: (0, 0)),
        ],
        out_shape=[
            jax.ShapeDtypeStruct((T, K), jnp.int32),
            jax.ShapeDtypeStruct((T, K), jnp.int32),
            jax.ShapeDtypeStruct((T, K), jnp.float32),
            jax.ShapeDtypeStruct((T, XC), jnp.int32),
            jax.ShapeDtypeStruct((1, 2 * E), jnp.int32),
        ],
        scratch_shapes=[pltpu.VMEM((1, E), jnp.float32)],
    )(x, gw)


_SCAN_UNROLL = 4


def _dispatch_body(s0_hbm, s1_hbm, counts_hbm, x_hbm, xe_hbm,
                   slots_v, tok_v, cnt_v,
                   b0, b1, b2, g0, g1, g2, w0, w1, w2):
    wid = lax.axis_index("s") * NC + lax.axis_index("c")
    base = wid * ROWS_PER_W

    pltpu.sync_copy(counts_hbm.at[0], cnt_v)
    iot = lax.iota(jnp.int32, LANES)
    for j in range(ROWS_PER_W // LANES):
        tok_v[pl.ds(j * LANES, LANES)] = jnp.zeros((LANES,), jnp.int32)

    for s_hbm in (s0_hbm, s1_hbm):
        pltpu.sync_copy(s_hbm, slots_v)

        def scan(j, carry):
            for u in range(_SCAN_UNROLL):
                off = j * (LANES * _SCAN_UNROLL) + u * LANES
                idx = slots_v[pl.ds(off, LANES)]
                tok = off + iot
                loc = idx - base
                mask = (loc >= 0) & (loc < ROWS_PER_W)
                locc = jnp.minimum(jnp.maximum(loc, 0), ROWS_PER_W - 1)
                plsc.store_scatter(tok_v, [locc], tok, mask=mask)
            return carry

        lax.fori_loop(0, T // (LANES * _SCAN_UNROLL), scan, 0)

    cnts = cnt_v[pl.ds(0, LANES)]
    my_e = wid // (CAP // ROWS_PER_W)
    my_cnt = jnp.sum(jnp.where(lax.iota(jnp.int32, LANES) == my_e, cnts, 0))
    nvalid = my_cnt - (wid % (CAP // ROWS_PER_W)) * ROWS_PER_W
    _pipelined_gather(x_hbm, tok_v, xe_hbm, base, ROWS_PER_W,
                      (b0, b1, b2), (g0, g1, g2), (w0, w1, w2),
                      cond=lambda cc: cc * RC < nvalid)


def _dispatch_gather(slots0, slots1, counts, xpack):
    mesh = plsc.VectorSubcoreMesh(
        core_axis_name="c", subcore_axis_name="s",
        num_cores=NC, num_subcores=NS)
    return pl.kernel(
        _dispatch_body,
        out_type=jax.ShapeDtypeStruct((EC, XC), jnp.int32),
        mesh=mesh,
        compiler_params=pltpu.CompilerParams(needs_layout_passes=False),
        scratch_types=[
            pltpu.VMEM((T,), jnp.int32),
            pltpu.VMEM((ROWS_PER_W,), jnp.int32),
            pltpu.VMEM((2 * E,), jnp.int32),
        ] + [pltpu.VMEM((RC, XC), jnp.int32)] * NBUF
          + [pltpu.SemaphoreType.DMA] * (2 * NBUF),
    )(slots0, slots1, counts, xpack)


BF1 = 1024
BD2 = 1024


def _ffn1_body(xe_ref, wg_ref, wu_ref, h_ref):
    xb = _unpack_pairs(xe_ref[0]).astype(jnp.float32)
    g = jnp.dot(xb, wg_ref[0], preferred_element_type=jnp.float32)
    u = jnp.dot(xb, wu_ref[0], preferred_element_type=jnp.float32)
    h_ref[0] = (g * jax.nn.sigmoid(g) * u).astype(jnp.bfloat16)


def _ffn1(xe3, w_gate, w_up):
    return pl.pallas_call(
        _ffn1_body,
        grid=(E, DFF // BF1),
        in_specs=[
            pl.BlockSpec((1, CAP, XC), lambda e, f: (e, 0, 0)),
            pl.BlockSpec((1, D, BF1), lambda e, f: (e, 0, f)),
            pl.BlockSpec((1, D, BF1), lambda e, f: (e, 0, f)),
        ],
        out_specs=pl.BlockSpec((1, CAP, BF1), lambda e, f: (e, 0, f)),
        out_shape=jax.ShapeDtypeStruct((E, CAP, DFF), jnp.bfloat16),
    )(xe3, w_gate, w_up)


def _ffn2_body(h_ref, wd_ref, ye_ref):
    yb = jnp.dot(h_ref[0].astype(jnp.float32), wd_ref[0],
                 preferred_element_type=jnp.float32).astype(jnp.bfloat16)
    ye_ref[0] = _pack_pairs(yb)


def _ffn2(h, w_down):
    return pl.pallas_call(
        _ffn2_body,
        grid=(E, D // BD2),
        in_specs=[
            pl.BlockSpec((1, CAP, DFF), lambda e, d: (e, 0, 0)),
            pl.BlockSpec((1, DFF, BD2), lambda e, d: (e, 0, d)),
        ],
        out_specs=pl.BlockSpec((1, CAP, BD2 // 2), lambda e, d: (e, 0, d)),
        out_shape=jax.ShapeDtypeStruct((E, CAP, XC), jnp.int32),
    )(h, w_down)


def _combine_body(ye_hbm, s0_hbm, s1_hbm, yg0_hbm, yg1_hbm,
                  idx0_v, idx1_v, b0, b1, b2, g0, g1, g2, w0, w1, w2):
    wid = lax.axis_index("s") * NC + lax.axis_index("c")
    tbase = wid * TOK_PER_W

    for iv, s_hbm, o_hbm in ((idx0_v, s0_hbm, yg0_hbm),
                             (idx1_v, s1_hbm, yg1_hbm)):
        pltpu.sync_copy(s_hbm.at[pl.ds(tbase, TOK_PER_W)], iv)
        _pipelined_gather(ye_hbm, iv, o_hbm, tbase, TOK_PER_W,
                          (b0, b1, b2), (g0, g1, g2), (w0, w1, w2))


def _combine_gather(ye, slots0, slots1):
    mesh = plsc.VectorSubcoreMesh(
        core_axis_name="c", subcore_axis_name="s",
        num_cores=NC, num_subcores=NS)
    return pl.kernel(
        _combine_body,
        out_type=[
            jax.ShapeDtypeStruct((T, XC), jnp.int32),
            jax.ShapeDtypeStruct((T, XC), jnp.int32),
        ],
        mesh=mesh,
        compiler_params=pltpu.CompilerParams(needs_layout_passes=False),
        scratch_types=[
            pltpu.VMEM((TOK_PER_W,), jnp.int32),
            pltpu.VMEM((TOK_PER_W,), jnp.int32),
        ] + [pltpu.VMEM((RC, XC), jnp.int32)] * NBUF
          + [pltpu.SemaphoreType.DMA] * (2 * NBUF),
    )(ye, slots0, slots1)


def _shared1_body(x_ref, wsg_ref, wsu_ref, hsh_ref):
    xb = x_ref[...]
    g = jnp.dot(xb, wsg_ref[...], preferred_element_type=jnp.float32)
    u = jnp.dot(xb, wsu_ref[...], preferred_element_type=jnp.float32)
    hsh_ref[...] = (g * jax.nn.sigmoid(g) * u).astype(jnp.bfloat16)


def _shared1(x, ws_gate, ws_up):
    return pl.pallas_call(
        _shared1_body,
        grid=(T // BT,),
        in_specs=[
            pl.BlockSpec((BT, D), lambda i: (i, 0)),
            pl.BlockSpec((D, DSH), lambda i: (0, 0)),
            pl.BlockSpec((D, DSH), lambda i: (0, 0)),
        ],
        out_specs=pl.BlockSpec((BT, DSH), lambda i: (i, 0)),
        out_shape=jax.ShapeDtypeStruct((T, DSH), jnp.bfloat16),
    )(x, ws_gate, ws_up)


def _unpack_ye_row(pi):
    n = BD2 // 2
    return jnp.concatenate(
        [_unpack_pairs(pi[:, d * n:(d + 1) * n]) for d in range(D // BD2)],
        axis=-1)


def _final_body(hsh_ref, wsd_ref, yg0_ref, yg1_ref, ws_ref, y_ref):
    ysh = jnp.dot(hsh_ref[...].astype(jnp.float32), wsd_ref[...],
                  preferred_element_type=jnp.float32)
    w0 = ws_ref[:, 0:1]
    w1 = ws_ref[:, 1:2]
    y_ref[...] = (ysh + w0 * _unpack_ye_row(yg0_ref[...]).astype(jnp.float32)
                  + w1 * _unpack_ye_row(yg1_ref[...]).astype(jnp.float32))


def _final(hsh, ws_down, yg0, yg1, ws):
    return pl.pallas_call(
        _final_body,
        grid=(T // BT,),
        in_specs=[
            pl.BlockSpec((BT, DSH), lambda i: (i, 0)),
            pl.BlockSpec((DSH, D), lambda i: (0, 0)),
            pl.BlockSpec((BT, XC), lambda i: (i, 0)),
            pl.BlockSpec((BT, XC), lambda i: (i, 0)),
            pl.BlockSpec((BT, K), lambda i: (i, 0)),
        ],
        out_specs=pl.BlockSpec((BT, D), lambda i: (i, 0)),
        out_shape=jax.ShapeDtypeStruct((T, D), jnp.float32),
    )(hsh, ws_down, yg0, yg1, ws)


def kernel(x, gate_w, w_gate, w_up, w_down, ws_gate, ws_up, ws_down):
    slots, slotsc, ws, xpack, counts = _router(x, gate_w)
    xe = _dispatch_gather(slots[:, 0], slots[:, 1], counts, xpack)
    hsh = _shared1(x, ws_gate, ws_up)
    h = _ffn1(xe.reshape(E, CAP, XC), w_gate, w_up)
    ye = _ffn2(h, w_down)
    yg0, yg1 = _combine_gather(ye.reshape(EC, XC),
                               slotsc[:, 0], slotsc[:, 1])
    return _final(hsh, ws_down, yg0, yg1, ws)
